# Initial kernel scaffold; baseline (speedup 1.0000x reference)
import jax
import jax.numpy as jnp
from jax.experimental import pallas as pl


def kernel(input_ids, kg_enc_input, cross_attn, last_hidden_state, entity_emb, rel_emb, W_mlp, b_mlp, W_lin, W_li, Wq, Wk, Wv, Wo, W_out, Wg, bg, Wc, bc):
    raise NotImplementedError("write your pallas kernel here")


# speedup vs baseline: 1.8256x; 1.8256x over previous
import functools

import jax
import jax.numpy as jnp
from jax import lax
from jax.experimental import pallas as pl
from jax.experimental.pallas import tpu as pltpu
from jax.experimental.pallas import tpu_sc as plsc

B = 8
MAX_LEN = 64
SRC_LEN = 128
NT = 500
NTP = 512
VOCAB = 50000
TE = 300
HIDDEN = 768
HEADS = 8
DK = 96
CH = 2048
NCH = 25
SVW = SRC_LEN + NTP

_NC, _NS = 2, 16
_NW = _NC * _NS
_RPW = (B * NTP) // _NW

f32 = jnp.float32


def _sc_gather_body(hidx, ridx, tidx, ent, rel, out_h, out_r, out_t,
                    idx_v, rows_v, sem):
    wid = lax.axis_index("s") * _NC + lax.axis_index("c")
    base = wid * _RPW
    pltpu.sync_copy(hidx.at[pl.ds(base, _RPW)], idx_v)
    pltpu.async_copy(ent.at[idx_v], rows_v, sem).wait()
    pltpu.sync_copy(rows_v, out_h.at[pl.ds(base, _RPW)])
    pltpu.sync_copy(ridx.at[pl.ds(base, _RPW)], idx_v)
    pltpu.async_copy(rel.at[idx_v], rows_v, sem).wait()
    pltpu.sync_copy(rows_v, out_r.at[pl.ds(base, _RPW)])
    pltpu.sync_copy(tidx.at[pl.ds(base, _RPW)], idx_v)
    pltpu.async_copy(ent.at[idx_v], rows_v, sem).wait()
    pltpu.sync_copy(rows_v, out_t.at[pl.ds(base, _RPW)])


def _sc_gather(head_i, rel_i, tail_i, entity_emb, rel_emb):
    call = functools.partial(
        pl.kernel,
        mesh=plsc.VectorSubcoreMesh(core_axis_name="c", subcore_axis_name="s"),
        out_type=[jax.ShapeDtypeStruct((B * NTP, TE), f32)] * 3,
        scratch_types=[
            pltpu.VMEM((_RPW,), jnp.int32),
            pltpu.VMEM((_RPW, TE), f32),
            pltpu.SemaphoreType.DMA,
        ],
        compiler_params=pltpu.CompilerParams(use_tc_tiling_on_sc=False),
    )(_sc_gather_body)
    return call(head_i, rel_i, tail_i, entity_emb, rel_emb)


def _wg_body(w_ref, g_ref, o_ref):
    c = pl.program_id(0)
    col = c * CH + lax.broadcasted_iota(jnp.int32, (1, CH), 1)
    valid = col < VOCAB
    g = jnp.where(valid, g_ref[...], 0.0)
    w = jnp.where(valid, w_ref[...], 0.0)
    contrib = lax.dot_general(g, w, (((1,), (1,)), ((), ())),
                              preferred_element_type=f32)

    @pl.when(c == 0)
    def _():
        o_ref[...] = jnp.zeros_like(o_ref)

    o_ref[...] += contrib


def _wg_pass(W_out, Wg_row):
    return pl.pallas_call(
        _wg_body,
        grid=(NCH,),
        in_specs=[
            pl.BlockSpec((HIDDEN, CH), lambda c: (0, c)),
            pl.BlockSpec((1, CH), lambda c: (0, c)),
        ],
        out_specs=pl.BlockSpec((1, HIDDEN), lambda c: (0, 0)),
        out_shape=jax.ShapeDtypeStruct((1, HIDDEN), f32),
    )(W_out, Wg_row)


def _prepass_body(h_ref, r_ref, t_ref, lhs_ref, cross_ref,
                  wmh_ref, wmr_ref, wmt_ref, bm_ref, wlin_ref, wli_ref,
                  wq_ref, wk_ref, wv_ref, wo_ref,
                  wge_ref, wc_ref, bg_ref, bc_ref,
                  soh_ref, sval_ref):
    tm = (jnp.dot(h_ref[...], wmh_ref[...], preferred_element_type=f32)
          + jnp.dot(r_ref[...], wmr_ref[...], preferred_element_type=f32)
          + jnp.dot(t_ref[...], wmt_ref[...], preferred_element_type=f32)
          + bm_ref[...])
    trip = jnp.dot(tm, wlin_ref[...], preferred_element_type=f32)
    out_h = jnp.dot(lhs_ref[0], wli_ref[...], preferred_element_type=f32)

    key_pos = lax.broadcasted_iota(jnp.int32, (MAX_LEN, NTP), 1)
    key_bias = jnp.where(key_pos < NT, 0.0, -1e30)
    scale = 1.0 / (DK ** 0.5)

    attn_acc = jnp.zeros((MAX_LEN, NTP), f32)
    mid = jnp.zeros((MAX_LEN, HIDDEN), f32)
    for hd in range(HEADS):
        q = jnp.dot(out_h, wq_ref[hd], preferred_element_type=f32)
        k = jnp.dot(trip, wk_ref[hd], preferred_element_type=f32)
        v = jnp.dot(trip, wv_ref[hd], preferred_element_type=f32)
        s = lax.dot_general(q, k, (((1,), (1,)), ((), ())),
                            preferred_element_type=f32) * scale
        s = s + key_bias
        m = jnp.max(s, axis=1, keepdims=True)
        e = jnp.exp(s - m)
        p = e / jnp.sum(e, axis=1, keepdims=True)
        attn_acc += p
        ctx = jnp.dot(p, v, preferred_element_type=f32)
        mid += jnp.dot(ctx, wo_ref[hd], preferred_element_type=f32)

    attn_mean = attn_acc * (1.0 / HEADS)
    dlg = jnp.mean(cross_ref[0], axis=0)

    p_gen = jax.nn.sigmoid(
        jnp.sum(out_h * wge_ref[...], axis=1, keepdims=True) + bg_ref[0, 0])
    p_con = jax.nn.sigmoid(
        jnp.sum(mid * wc_ref[...], axis=1, keepdims=True) + bc_ref[0, 0])

    soh_ref[...] = (out_h * ((1.0 - p_con) * p_gen))[None]
    sd = dlg * ((1.0 - p_con) * (1.0 - p_gen))
    sa = attn_mean * p_con
    sval_ref[...] = jnp.concatenate([sd, sa], axis=1)[None]


def _prepass(hrows, rrows, trows, lhs, cross, Wmh, Wmr, Wmt, bm2, W_lin,
             W_li, Wq_r, Wk_r, Wv_r, Wo_r, wge_row, Wc_row, bg2, bc2):
    full = lambda shape: pl.BlockSpec(shape, lambda b: (0,) * len(shape))
    return pl.pallas_call(
        _prepass_body,
        grid=(B,),
        in_specs=[
            pl.BlockSpec((NTP, TE), lambda b: (b, 0)),
            pl.BlockSpec((NTP, TE), lambda b: (b, 0)),
            pl.BlockSpec((NTP, TE), lambda b: (b, 0)),
            pl.BlockSpec((1, MAX_LEN, 2 * HIDDEN), lambda b: (b, 0, 0)),
            pl.BlockSpec((1, 12, MAX_LEN, SRC_LEN), lambda b: (b, 0, 0, 0)),
            full((TE, 3 * TE)),
            full((TE, 3 * TE)),
            full((TE, 3 * TE)),
            full((1, 3 * TE)),
            full((3 * TE, HIDDEN)),
            full((2 * HIDDEN, HIDDEN)),
            full((HEADS, HIDDEN, DK)),
            full((HEADS, HIDDEN, DK)),
            full((HEADS, HIDDEN, DK)),
            full((HEADS, DK, HIDDEN)),
            full((1, HIDDEN)),
            full((1, HIDDEN)),
            full((1, 1)),
            full((1, 1)),
        ],
        out_specs=[
            pl.BlockSpec((1, MAX_LEN, HIDDEN), lambda b: (b, 0, 0)),
            pl.BlockSpec((1, MAX_LEN, SVW), lambda b: (b, 0, 0)),
        ],
        out_shape=[
            jax.ShapeDtypeStruct((B, MAX_LEN, HIDDEN), f32),
            jax.ShapeDtypeStruct((B, MAX_LEN, SVW), f32),
        ],
    )(hrows, rrows, trows, lhs, cross, Wmh, Wmr, Wmt, bm2, W_lin, W_li,
      Wq_r, Wk_r, Wv_r, Wo_r, wge_row, Wc_row, bg2, bc2)


def _vocab_body(soh_ref, sval_ref, idx_ref, w_ref, o_ref):
    c = pl.program_id(0)
    gen = jnp.dot(soh_ref[...], w_ref[...], preferred_element_type=f32)
    gen3 = gen.reshape(B, MAX_LEN, CH)
    col = c * CH + lax.broadcasted_iota(jnp.int32, (SVW, CH), 1)
    for b in range(B):
        oh = (idx_ref[b] == col).astype(f32)
        cp = jnp.dot(sval_ref[b], oh, preferred_element_type=f32)
        o_ref[b] = gen3[b] + cp


def _vocab_pass(soh2, svals, comb3, W_out):
    return pl.pallas_call(
        _vocab_body,
        grid=(NCH,),
        in_specs=[
            pl.BlockSpec((B * MAX_LEN, HIDDEN), lambda c: (0, 0)),
            pl.BlockSpec((B, MAX_LEN, SVW), lambda c: (0, 0, 0)),
            pl.BlockSpec((B, SVW, 1), lambda c: (0, 0, 0)),
            pl.BlockSpec((HIDDEN, CH), lambda c: (0, c)),
        ],
        out_specs=pl.BlockSpec((B, MAX_LEN, CH), lambda c: (0, 0, c)),
        out_shape=jax.ShapeDtypeStruct((B, MAX_LEN, VOCAB), f32),
    )(soh2, svals, comb3, W_out)


def kernel(input_ids, kg_enc_input, cross_attn, last_hidden_state,
           entity_emb, rel_emb, W_mlp, b_mlp, W_lin, W_li, Wq, Wk, Wv, Wo,
           W_out, Wg, bg, Wc, bc):
    kg = kg_enc_input.reshape(B, NT, 3)
    pad0 = jnp.zeros((B, NTP - NT), jnp.int32)
    head_i = jnp.concatenate([kg[:, :, 0], pad0], axis=1).reshape(B * NTP)
    rel_i = jnp.concatenate([kg[:, :, 1], pad0], axis=1).reshape(B * NTP)
    tail_i = jnp.concatenate([kg[:, :, 2], pad0], axis=1).reshape(B * NTP)

    hrows, rrows, trows = _sc_gather(head_i, rel_i, tail_i,
                                     entity_emb, rel_emb)

    wge_row = _wg_pass(W_out, Wg.reshape(1, VOCAB))

    Wmh, Wmr, Wmt = W_mlp[:TE], W_mlp[TE:2 * TE], W_mlp[2 * TE:]
    Wq_r = Wq.reshape(HIDDEN, HEADS, DK).transpose(1, 0, 2)
    Wk_r = Wk.reshape(HIDDEN, HEADS, DK).transpose(1, 0, 2)
    Wv_r = Wv.reshape(HIDDEN, HEADS, DK).transpose(1, 0, 2)
    Wo_r = Wo.reshape(HEADS, DK, HIDDEN)

    soh, svals = _prepass(
        hrows, rrows, trows, last_hidden_state, cross_attn,
        Wmh, Wmr, Wmt, b_mlp.reshape(1, 3 * TE), W_lin, W_li,
        Wq_r, Wk_r, Wv_r, Wo_r, wge_row, Wc.reshape(1, HIDDEN),
        bg.reshape(1, 1), bc.reshape(1, 1))

    padm1 = jnp.full((B, NTP - NT), -1, jnp.int32)
    tail_pad = jnp.concatenate([kg[:, :, 2], padm1], axis=1)
    comb3 = jnp.concatenate([input_ids, tail_pad], axis=1)[..., None]

    return _vocab_pass(soh.reshape(B * MAX_LEN, HIDDEN), svals, comb3, W_out)
